# baseline (speedup 1.0000x reference)
import functools

import jax
import jax.numpy as jnp
from jax import lax
from jax.experimental import pallas as pl
from jax.experimental.pallas import tpu as pltpu
from jax.experimental.pallas import tpu_sc as plsc

NN = 4
DIM = 768
E = 8
TOPK = 2
EXPERT_DIM = 3072
SEQ = 2048
DC = DIM // NN
FC = EXPERT_DIM // NN

TN = 1024
NT = SEQ // TN

T = 128
NA = SEQ * TOPK
PS = 5120
NTILES = PS // T
NTP = 48

NC = 2
NS = 16
NW = NC * NS
SPW = PS // NW
L = 16


def _router_body(x_ref, wrt_ref, br_ref, tidx_ref, tw_ref):
    logits = lax.dot_general(
        x_ref[...], wrt_ref[...], (((1,), (0,)), ((), ())),
        preferred_element_type=jnp.float32) + br_ref[...]
    tn = logits.shape[0]
    iota = lax.broadcasted_iota(jnp.int32, (tn, E), 1)
    m1 = jnp.max(logits, axis=1, keepdims=True)
    am1 = jnp.min(jnp.where(logits == m1, iota, E), axis=1, keepdims=True)
    masked = jnp.where(iota == am1, -jnp.inf, logits)
    m2 = jnp.max(masked, axis=1, keepdims=True)
    am2 = jnp.min(jnp.where(masked == m2, iota, E), axis=1, keepdims=True)
    w1 = 1.0 / (1.0 + jnp.exp(m2 - m1))
    w2 = 1.0 - w1
    tidx_ref[...] = jnp.concatenate([am1, am2], axis=1)
    tw_ref[...] = jnp.concatenate([w1, w2], axis=1)


def _dispatch_body(tidx_hbm, tw_hbm, st_hbm, sg_hbm, te_hbm, sa_hbm,
                   ids_v, w_v, buf_t, buf_g, buf_j, te_v):
    wid = lax.axis_index("s") * NC + lax.axis_index("c")
    lanes = lax.broadcasted_iota(jnp.int32, (L,), 0)
    zi = jnp.zeros((L,), jnp.int32)
    zf = jnp.zeros((L,), jnp.float32)

    pltpu.sync_copy(tidx_hbm, ids_v)

    def cbody(ch, cs):
        v = ids_v[pl.ds(ch * L, L)]
        return tuple(c + jnp.where(v == e, 1, 0) for e, c in enumerate(cs))

    cvecs = lax.fori_loop(0, NA // L, cbody, tuple(zi for _ in range(E)))
    cnt = [jnp.sum(cv) for cv in cvecs]
    pe = [((c + T - 1) // T) * T for c in cnt]
    incl = []
    run = jnp.int32(0)
    for e in range(E):
        run = run + pe[e]
        incl.append(run)
    tot = incl[E - 1]

    @pl.when(wid < E)
    def _():
        pltpu.sync_copy(tw_hbm, w_v)
        mycnt = jnp.int32(0)
        base = jnp.int32(0)
        for e in range(E):
            mycnt = jnp.where(wid == e, cnt[e], mycnt)
            base = jnp.where(wid == e, incl[e] - pe[e], base)
        base = pl.multiple_of(base, T)

        def zbody(i, _):
            buf_t[pl.ds(i * L, L)] = zi
            buf_g[pl.ds(i * L, L)] = zf
            buf_j[pl.ds(i * L, L)] = zi + NA
            return 0

        lax.fori_loop(0, SEQ // L, zbody, 0)

        def pbody(ch, nxt):
            v = ids_v[pl.ds(ch * L, L)]
            m = v == wid
            mi = jnp.where(m, 1, 0)
            pr = plsc.cumsum(mi)
            pos = nxt + pr - 1
            j = ch * L + lanes
            plsc.store_scatter(buf_t, [pos], j >> 1, mask=m)
            plsc.store_scatter(buf_g, [pos], w_v[pl.ds(ch * L, L)], mask=m)
            plsc.store_scatter(buf_j, [pos], j, mask=m)
            return nxt + jnp.sum(mi)

        lax.fori_loop(0, NA // L, pbody, jnp.int32(0))

        for k in range(SEQ // T):
            @pl.when(k * T < mycnt)
            def _():
                pltpu.sync_copy(buf_t.at[pl.ds(k * T, T)],
                                st_hbm.at[pl.ds(base + k * T, T)])
                pltpu.sync_copy(buf_g.at[pl.ds(k * T, T)],
                                sg_hbm.at[pl.ds(base + k * T, T)])
                pltpu.sync_copy(buf_j.at[pl.ds(k * T, T)],
                                sa_hbm.at[pl.ds(base + k * T, T)])

    @pl.when(wid == E)
    def _():
        def fbody(i, _):
            buf_t[pl.ds(i * L, L)] = zi
            buf_g[pl.ds(i * L, L)] = zf
            buf_j[pl.ds(i * L, L)] = zi + NA
            return 0

        lax.fori_loop(0, T // L, fbody, 0)
        for k in range(NTILES):
            @pl.when(k * T >= tot)
            def _():
                pltpu.sync_copy(buf_t.at[pl.ds(0, T)],
                                st_hbm.at[pl.ds(k * T, T)])
                pltpu.sync_copy(buf_g.at[pl.ds(0, T)],
                                sg_hbm.at[pl.ds(k * T, T)])
                pltpu.sync_copy(buf_j.at[pl.ds(0, T)],
                                sa_hbm.at[pl.ds(k * T, T)])

    @pl.when(wid == E + 1)
    def _():
        for r in range(NTP // L):
            tstart = (lanes + r * L) * T
            acc = zi
            for e in range(E):
                acc = acc + jnp.where(tstart >= incl[e], 1, 0)
            te_v[pl.ds(r * L, L)] = jnp.minimum(acc, E - 1)
        pltpu.sync_copy(te_v, te_hbm)


NCH = 2
CH = SPW // NCH


def _gather_body(x_hbm, st_hbm, xs_hbm, idx_v, rows_a, rows_b,
                 gs_a, gs_b, ws_a, ws_b):
    wid = lax.axis_index("s") * NC + lax.axis_index("c")
    base = pl.multiple_of(wid * SPW, SPW)
    bufs = (rows_a, rows_b)
    gsems = (gs_a, gs_b)
    wsems = (ws_a, ws_b)
    pltpu.sync_copy(st_hbm.at[pl.ds(base, SPW)], idx_v)
    gd = [pltpu.async_copy(x_hbm.at[idx_v.at[pl.ds(k * CH, CH)]], bufs[k],
                           gsems[k]) for k in range(NCH)]
    wd = [None] * NCH
    for k in range(NCH):
        gd[k].wait()
        wd[k] = pltpu.async_copy(
            bufs[k], xs_hbm.at[pl.ds(base + k * CH, CH)], wsems[k])
    for k in range(NCH):
        wd[k].wait()


def _scatter_body(ys_hbm, sa_hbm, yg_hbm, aidx_v, rows_a, rows_b,
                  gs_a, gs_b, ws_a, ws_b):
    wid = lax.axis_index("s") * NC + lax.axis_index("c")
    base = pl.multiple_of(wid * SPW, SPW)
    bufs = (rows_a, rows_b)
    gsems = (gs_a, gs_b)
    wsems = (ws_a, ws_b)
    for k in range(NCH):
        pltpu.sync_copy(sa_hbm.at[pl.ds(base + k * CH, CH)], aidx_v.at[k])
    rd = [pltpu.async_copy(ys_hbm.at[pl.ds(base + k * CH, CH)], bufs[k],
                           gsems[k]) for k in range(NCH)]
    sd = [None] * NCH
    for k in range(NCH):
        rd[k].wait()
        sd[k] = pltpu.async_copy(bufs[k], yg_hbm.at[aidx_v.at[k]], wsems[k])
    for k in range(NCH):
        sd[k].wait()


def _gemm_body(te_sref, a1_ref, xs_ref, s1cat_ref, b1_ref, w2_ref, b2_ref,
               g_ref, ys_ref):
    x = xs_ref[...]
    s1cat = s1cat_ref[0]
    w2 = w2_ref[0]
    y = None
    for a in range(NN):
        ha = None
        for i in range(NN):
            xc = a1_ref[0, i, a, 0] * x[:, 0:DC]
            for b in range(1, NN):
                xc = xc + a1_ref[0, i, a, b] * x[:, b * DC:(b + 1) * DC]
            pp = lax.dot_general(
                xc, s1cat[i * DC:(i + 1) * DC, :], (((1,), (0,)), ((), ())),
                preferred_element_type=jnp.float32)
            ha = pp if ha is None else ha + pp
        ha = ha + b1_ref[0, 0, a * FC:(a + 1) * FC]
        ha = 0.5 * ha * (1.0 + lax.erf(ha * (2.0 ** -0.5)))
        ya = lax.dot_general(ha, w2[:, a * FC:(a + 1) * FC],
                             (((1,), (1,)), ((), ())),
                             preferred_element_type=jnp.float32)
        y = ya if y is None else y + ya
    ys_ref[...] = g_ref[...] * (y + b2_ref[0, 0])


def _combine_body(yg_ref, out_ref):
    out_ref[...] = yg_ref[:, :DIM] + yg_ref[:, DIM:]


@functools.lru_cache(maxsize=None)
def _sc_mesh():
    return plsc.VectorSubcoreMesh(core_axis_name="c", subcore_axis_name="s")


def _dispatch(tidx_flat, tw_flat):
    return pl.kernel(
        _dispatch_body,
        out_type=(jax.ShapeDtypeStruct((PS,), jnp.int32),
                  jax.ShapeDtypeStruct((PS,), jnp.float32),
                  jax.ShapeDtypeStruct((NTP,), jnp.int32),
                  jax.ShapeDtypeStruct((PS,), jnp.int32)),
        mesh=_sc_mesh(),
        compiler_params=pltpu.CompilerParams(needs_layout_passes=False),
        scratch_types=[
            pltpu.VMEM((NA,), jnp.int32),
            pltpu.VMEM((NA,), jnp.float32),
            pltpu.VMEM((SEQ,), jnp.int32),
            pltpu.VMEM((SEQ,), jnp.float32),
            pltpu.VMEM((SEQ,), jnp.int32),
            pltpu.VMEM((NTP,), jnp.int32),
        ],
    )(tidx_flat, tw_flat)


def _gather(x2d, st):
    return pl.kernel(
        _gather_body,
        out_type=jax.ShapeDtypeStruct((PS, DIM), jnp.float32),
        mesh=_sc_mesh(),
        compiler_params=pltpu.CompilerParams(needs_layout_passes=False),
        scratch_types=[
            pltpu.VMEM((SPW,), jnp.int32),
            pltpu.VMEM((CH, DIM), jnp.float32),
            pltpu.VMEM((CH, DIM), jnp.float32),
            pltpu.SemaphoreType.DMA,
            pltpu.SemaphoreType.DMA,
            pltpu.SemaphoreType.DMA,
            pltpu.SemaphoreType.DMA,
        ],
    )(x2d, st)


def _scatter(ys, sa):
    return pl.kernel(
        _scatter_body,
        out_type=jax.ShapeDtypeStruct((NA + 1, DIM), jnp.float32),
        mesh=_sc_mesh(),
        compiler_params=pltpu.CompilerParams(needs_layout_passes=False),
        scratch_types=[
            pltpu.VMEM((NCH, CH), jnp.int32),
            pltpu.VMEM((CH, DIM), jnp.float32),
            pltpu.VMEM((CH, DIM), jnp.float32),
            pltpu.SemaphoreType.DMA,
            pltpu.SemaphoreType.DMA,
            pltpu.SemaphoreType.DMA,
            pltpu.SemaphoreType.DMA,
        ],
    )(ys, sa)


def kernel(x, A_r, S_r, b_r, A1, S1, b1, W2, b2):
    x2d = x.reshape(SEQ, DIM)
    wr = jnp.sum(
        jnp.einsum('iab,icd->iacbd', A_r, S_r).reshape(NN, E, DIM), axis=0)
    tidx, tw = pl.pallas_call(
        _router_body,
        grid=(NT,),
        in_specs=[
            pl.BlockSpec((TN, DIM), lambda t: (t, 0)),
            pl.BlockSpec((DIM, E), lambda t: (0, 0)),
            pl.BlockSpec((1, E), lambda t: (0, 0)),
        ],
        out_specs=[
            pl.BlockSpec((TN, TOPK), lambda t: (t, 0)),
            pl.BlockSpec((TN, TOPK), lambda t: (t, 0)),
        ],
        out_shape=[
            jax.ShapeDtypeStruct((SEQ, TOPK), jnp.int32),
            jax.ShapeDtypeStruct((SEQ, TOPK), jnp.float32),
        ],
    )(x2d, wr.T, b_r.reshape(1, E))

    st, sg, te, sa = _dispatch(tidx.reshape(NA), tw.reshape(NA))
    xs = _gather(x2d, st)

    s1cat = jnp.transpose(S1, (0, 1, 3, 2)).reshape(E, DIM, FC)
    grid_spec = pltpu.PrefetchScalarGridSpec(
        num_scalar_prefetch=1,
        grid=(NTILES,),
        in_specs=[
            pl.BlockSpec((1, NN, NN, NN), lambda t, te_r: (te_r[t], 0, 0, 0),
                         memory_space=pltpu.SMEM),
            pl.BlockSpec((T, DIM), lambda t, te_r: (t, 0)),
            pl.BlockSpec((1, DIM, FC), lambda t, te_r: (te_r[t], 0, 0)),
            pl.BlockSpec((1, 1, EXPERT_DIM), lambda t, te_r: (te_r[t], 0, 0)),
            pl.BlockSpec((1, DIM, EXPERT_DIM),
                         lambda t, te_r: (te_r[t], 0, 0)),
            pl.BlockSpec((1, 1, DIM), lambda t, te_r: (te_r[t], 0, 0)),
            pl.BlockSpec((T, 1), lambda t, te_r: (t, 0)),
        ],
        out_specs=pl.BlockSpec((T, DIM), lambda t, te_r: (t, 0)),
    )
    ys = pl.pallas_call(
        _gemm_body,
        grid_spec=grid_spec,
        out_shape=jax.ShapeDtypeStruct((PS, DIM), jnp.float32),
    )(te, A1, xs, s1cat, b1.reshape(E, 1, EXPERT_DIM), W2,
      b2.reshape(E, 1, DIM), sg.reshape(PS, 1))

    yg = _scatter(ys, sa)
    yg_m = yg[:NA].reshape(SEQ, TOPK * DIM)
    out2d = pl.pallas_call(
        _combine_body,
        grid=(NT,),
        in_specs=[pl.BlockSpec((TN, TOPK * DIM), lambda t: (t, 0))],
        out_specs=pl.BlockSpec((TN, DIM), lambda t: (t, 0)),
        out_shape=jax.ShapeDtypeStruct((SEQ, DIM), jnp.float32),
    )(yg_m)
    return out2d.reshape(x.shape)


# speedup vs baseline: 1.0094x; 1.0094x over previous
import functools

import jax
import jax.numpy as jnp
from jax import lax
from jax.experimental import pallas as pl
from jax.experimental.pallas import tpu as pltpu
from jax.experimental.pallas import tpu_sc as plsc

NN = 4
DIM = 768
E = 8
TOPK = 2
EXPERT_DIM = 3072
SEQ = 2048
DC = DIM // NN
FC = EXPERT_DIM // NN

TN = 1024
NT = SEQ // TN

T = 128
NA = SEQ * TOPK
PS = 5120
NTILES = PS // T
NTP = 48

NC = 2
NS = 16
NW = NC * NS
SPW = PS // NW
L = 16


def _router_body(x_ref, wrt_ref, br_ref, tidx_ref, tw_ref):
    logits = lax.dot_general(
        x_ref[...], wrt_ref[...], (((1,), (0,)), ((), ())),
        preferred_element_type=jnp.float32) + br_ref[...]
    tn = logits.shape[0]
    iota = lax.broadcasted_iota(jnp.int32, (tn, E), 1)
    m1 = jnp.max(logits, axis=1, keepdims=True)
    am1 = jnp.min(jnp.where(logits == m1, iota, E), axis=1, keepdims=True)
    masked = jnp.where(iota == am1, -jnp.inf, logits)
    m2 = jnp.max(masked, axis=1, keepdims=True)
    am2 = jnp.min(jnp.where(masked == m2, iota, E), axis=1, keepdims=True)
    w1 = 1.0 / (1.0 + jnp.exp(m2 - m1))
    w2 = 1.0 - w1
    tidx_ref[...] = jnp.concatenate([am1, am2], axis=1)
    tw_ref[...] = jnp.concatenate([w1, w2], axis=1)


def _dispatch_body(tidx_hbm, tw_hbm, st_hbm, sg_hbm, te_hbm, sa_hbm,
                   ids_v, w_v, buf_t, buf_g, buf_j, te_v):
    wid = lax.axis_index("s") * NC + lax.axis_index("c")
    lanes = lax.broadcasted_iota(jnp.int32, (L,), 0)
    zi = jnp.zeros((L,), jnp.int32)
    zf = jnp.zeros((L,), jnp.float32)

    pltpu.sync_copy(tidx_hbm, ids_v)

    def cbody(ch, cs):
        v = ids_v[pl.ds(ch * L, L)]
        return tuple(c + jnp.where(v == e, 1, 0) for e, c in enumerate(cs))

    cvecs = lax.fori_loop(0, NA // L, cbody, tuple(zi for _ in range(E)))
    cnt = [jnp.sum(cv) for cv in cvecs]
    pe = [((c + T - 1) // T) * T for c in cnt]
    incl = []
    run = jnp.int32(0)
    for e in range(E):
        run = run + pe[e]
        incl.append(run)
    tot = incl[E - 1]

    @pl.when(wid < E)
    def _():
        pltpu.sync_copy(tw_hbm, w_v)
        mycnt = jnp.int32(0)
        base = jnp.int32(0)
        for e in range(E):
            mycnt = jnp.where(wid == e, cnt[e], mycnt)
            base = jnp.where(wid == e, incl[e] - pe[e], base)
        base = pl.multiple_of(base, T)

        def zbody(i, _):
            buf_t[pl.ds(i * L, L)] = zi
            buf_g[pl.ds(i * L, L)] = zf
            buf_j[pl.ds(i * L, L)] = zi + NA
            return 0

        lax.fori_loop(0, SEQ // L, zbody, 0)

        def pbody(ch, nxt):
            v = ids_v[pl.ds(ch * L, L)]
            m = v == wid
            mi = jnp.where(m, 1, 0)
            pr = plsc.cumsum(mi)
            pos = nxt + pr - 1
            j = ch * L + lanes
            plsc.store_scatter(buf_t, [pos], j >> 1, mask=m)
            plsc.store_scatter(buf_g, [pos], w_v[pl.ds(ch * L, L)], mask=m)
            plsc.store_scatter(buf_j, [pos], j, mask=m)
            return nxt + jnp.sum(mi)

        lax.fori_loop(0, NA // L, pbody, jnp.int32(0))

        for k in range(SEQ // T):
            @pl.when(k * T < mycnt)
            def _():
                pltpu.sync_copy(buf_t.at[pl.ds(k * T, T)],
                                st_hbm.at[pl.ds(base + k * T, T)])
                pltpu.sync_copy(buf_g.at[pl.ds(k * T, T)],
                                sg_hbm.at[pl.ds(base + k * T, T)])
                pltpu.sync_copy(buf_j.at[pl.ds(k * T, T)],
                                sa_hbm.at[pl.ds(base + k * T, T)])

    @pl.when(wid == E)
    def _():
        def fbody(i, _):
            buf_t[pl.ds(i * L, L)] = zi
            buf_g[pl.ds(i * L, L)] = zf
            buf_j[pl.ds(i * L, L)] = zi + NA
            return 0

        lax.fori_loop(0, T // L, fbody, 0)
        for k in range(NTILES):
            @pl.when(k * T >= tot)
            def _():
                pltpu.sync_copy(buf_t.at[pl.ds(0, T)],
                                st_hbm.at[pl.ds(k * T, T)])
                pltpu.sync_copy(buf_g.at[pl.ds(0, T)],
                                sg_hbm.at[pl.ds(k * T, T)])
                pltpu.sync_copy(buf_j.at[pl.ds(0, T)],
                                sa_hbm.at[pl.ds(k * T, T)])

    @pl.when(wid == E + 1)
    def _():
        for r in range(NTP // L):
            tstart = (lanes + r * L) * T
            acc = zi
            for e in range(E):
                acc = acc + jnp.where(tstart >= incl[e], 1, 0)
            te_v[pl.ds(r * L, L)] = jnp.minimum(acc, E - 1)
        pltpu.sync_copy(te_v, te_hbm)


NCH = 2
CH = SPW // NCH


def _gather_body(x_hbm, st_hbm, xs_hbm, idx_v, rows_a, rows_b,
                 gs_a, gs_b, ws_a, ws_b):
    wid = lax.axis_index("s") * NC + lax.axis_index("c")
    base = pl.multiple_of(wid * SPW, SPW)
    bufs = (rows_a, rows_b)
    gsems = (gs_a, gs_b)
    wsems = (ws_a, ws_b)
    pltpu.sync_copy(st_hbm.at[pl.ds(base, SPW)], idx_v)
    gd = [pltpu.async_copy(x_hbm.at[idx_v.at[pl.ds(k * CH, CH)]], bufs[k],
                           gsems[k]) for k in range(NCH)]
    wd = [None] * NCH
    for k in range(NCH):
        gd[k].wait()
        wd[k] = pltpu.async_copy(
            bufs[k], xs_hbm.at[pl.ds(base + k * CH, CH)], wsems[k])
    for k in range(NCH):
        wd[k].wait()


def _scatter_body(ys_hbm, sa_hbm, yg_hbm, aidx_v, rows_a, rows_b,
                  gs_a, gs_b, ws_a, ws_b):
    wid = lax.axis_index("s") * NC + lax.axis_index("c")
    base = pl.multiple_of(wid * SPW, SPW)
    bufs = (rows_a, rows_b)
    gsems = (gs_a, gs_b)
    wsems = (ws_a, ws_b)
    for k in range(NCH):
        pltpu.sync_copy(sa_hbm.at[pl.ds(base + k * CH, CH)], aidx_v.at[k])
    rd = [pltpu.async_copy(ys_hbm.at[pl.ds(base + k * CH, CH)], bufs[k],
                           gsems[k]) for k in range(NCH)]
    sd = [None] * NCH
    for k in range(NCH):
        rd[k].wait()
        sd[k] = pltpu.async_copy(bufs[k], yg_hbm.at[aidx_v.at[k]], wsems[k])
    for k in range(NCH):
        sd[k].wait()


def _gemm_body(te_sref, a1_ref, xs_ref, s1cat_ref, b1_ref, w2_ref, b2_ref,
               g_ref, ys_ref):
    x = xs_ref[...]
    s1cat = s1cat_ref[0]
    h_blocks = []
    for a in range(NN):
        xc_parts = []
        for i in range(NN):
            acc = a1_ref[0, i, a, 0] * x[:, 0:DC]
            for b in range(1, NN):
                acc = acc + a1_ref[0, i, a, b] * x[:, b * DC:(b + 1) * DC]
            xc_parts.append(acc)
        xc = jnp.concatenate(xc_parts, axis=1)
        ha = lax.dot_general(xc, s1cat, (((1,), (0,)), ((), ())),
                             preferred_element_type=jnp.float32)
        h_blocks.append(ha)
    h = jnp.concatenate(h_blocks, axis=1) + b1_ref[0, 0]
    h = 0.5 * h * (1.0 + lax.erf(h * (2.0 ** -0.5)))
    y = lax.dot_general(h, w2_ref[0], (((1,), (1,)), ((), ())),
                        preferred_element_type=jnp.float32) + b2_ref[0, 0]
    ys_ref[...] = g_ref[...] * y


def _combine_body(yg_ref, out_ref):
    out_ref[...] = yg_ref[:, :DIM] + yg_ref[:, DIM:]


@functools.lru_cache(maxsize=None)
def _sc_mesh():
    return plsc.VectorSubcoreMesh(core_axis_name="c", subcore_axis_name="s")


def _dispatch(tidx_flat, tw_flat):
    return pl.kernel(
        _dispatch_body,
        out_type=(jax.ShapeDtypeStruct((PS,), jnp.int32),
                  jax.ShapeDtypeStruct((PS,), jnp.float32),
                  jax.ShapeDtypeStruct((NTP,), jnp.int32),
                  jax.ShapeDtypeStruct((PS,), jnp.int32)),
        mesh=_sc_mesh(),
        compiler_params=pltpu.CompilerParams(needs_layout_passes=False),
        scratch_types=[
            pltpu.VMEM((NA,), jnp.int32),
            pltpu.VMEM((NA,), jnp.float32),
            pltpu.VMEM((SEQ,), jnp.int32),
            pltpu.VMEM((SEQ,), jnp.float32),
            pltpu.VMEM((SEQ,), jnp.int32),
            pltpu.VMEM((NTP,), jnp.int32),
        ],
    )(tidx_flat, tw_flat)


def _gather(x2d, st):
    return pl.kernel(
        _gather_body,
        out_type=jax.ShapeDtypeStruct((PS, DIM), jnp.float32),
        mesh=_sc_mesh(),
        compiler_params=pltpu.CompilerParams(needs_layout_passes=False),
        scratch_types=[
            pltpu.VMEM((SPW,), jnp.int32),
            pltpu.VMEM((CH, DIM), jnp.float32),
            pltpu.VMEM((CH, DIM), jnp.float32),
            pltpu.SemaphoreType.DMA,
            pltpu.SemaphoreType.DMA,
            pltpu.SemaphoreType.DMA,
            pltpu.SemaphoreType.DMA,
        ],
    )(x2d, st)


def _scatter(ys, sa):
    return pl.kernel(
        _scatter_body,
        out_type=jax.ShapeDtypeStruct((NA + 1, DIM), jnp.float32),
        mesh=_sc_mesh(),
        compiler_params=pltpu.CompilerParams(needs_layout_passes=False),
        scratch_types=[
            pltpu.VMEM((NCH, CH), jnp.int32),
            pltpu.VMEM((CH, DIM), jnp.float32),
            pltpu.VMEM((CH, DIM), jnp.float32),
            pltpu.SemaphoreType.DMA,
            pltpu.SemaphoreType.DMA,
            pltpu.SemaphoreType.DMA,
            pltpu.SemaphoreType.DMA,
        ],
    )(ys, sa)


def kernel(x, A_r, S_r, b_r, A1, S1, b1, W2, b2):
    x2d = x.reshape(SEQ, DIM)
    wr = jnp.sum(
        jnp.einsum('iab,icd->iacbd', A_r, S_r).reshape(NN, E, DIM), axis=0)
    tidx, tw = pl.pallas_call(
        _router_body,
        grid=(NT,),
        in_specs=[
            pl.BlockSpec((TN, DIM), lambda t: (t, 0)),
            pl.BlockSpec((DIM, E), lambda t: (0, 0)),
            pl.BlockSpec((1, E), lambda t: (0, 0)),
        ],
        out_specs=[
            pl.BlockSpec((TN, TOPK), lambda t: (t, 0)),
            pl.BlockSpec((TN, TOPK), lambda t: (t, 0)),
        ],
        out_shape=[
            jax.ShapeDtypeStruct((SEQ, TOPK), jnp.int32),
            jax.ShapeDtypeStruct((SEQ, TOPK), jnp.float32),
        ],
    )(x2d, wr.T, b_r.reshape(1, E))

    st, sg, te, sa = _dispatch(tidx.reshape(NA), tw.reshape(NA))
    xs = _gather(x2d, st)

    s1cat = jnp.transpose(S1, (0, 1, 3, 2)).reshape(E, DIM, FC)
    grid_spec = pltpu.PrefetchScalarGridSpec(
        num_scalar_prefetch=1,
        grid=(NTILES,),
        in_specs=[
            pl.BlockSpec((1, NN, NN, NN), lambda t, te_r: (te_r[t], 0, 0, 0),
                         memory_space=pltpu.SMEM),
            pl.BlockSpec((T, DIM), lambda t, te_r: (t, 0)),
            pl.BlockSpec((1, DIM, FC), lambda t, te_r: (te_r[t], 0, 0)),
            pl.BlockSpec((1, 1, EXPERT_DIM), lambda t, te_r: (te_r[t], 0, 0)),
            pl.BlockSpec((1, DIM, EXPERT_DIM),
                         lambda t, te_r: (te_r[t], 0, 0)),
            pl.BlockSpec((1, 1, DIM), lambda t, te_r: (te_r[t], 0, 0)),
            pl.BlockSpec((T, 1), lambda t, te_r: (t, 0)),
        ],
        out_specs=pl.BlockSpec((T, DIM), lambda t, te_r: (t, 0)),
    )
    ys = pl.pallas_call(
        _gemm_body,
        grid_spec=grid_spec,
        out_shape=jax.ShapeDtypeStruct((PS, DIM), jnp.float32),
    )(te, A1, xs, s1cat, b1.reshape(E, 1, EXPERT_DIM), W2,
      b2.reshape(E, 1, DIM), sg.reshape(PS, 1))

    yg = _scatter(ys, sa)
    yg_m = yg[:NA].reshape(SEQ, TOPK * DIM)
    out2d = pl.pallas_call(
        _combine_body,
        grid=(NT,),
        in_specs=[pl.BlockSpec((TN, TOPK * DIM), lambda t: (t, 0))],
        out_specs=pl.BlockSpec((TN, DIM), lambda t: (t, 0)),
        out_shape=jax.ShapeDtypeStruct((SEQ, DIM), jnp.float32),
    )(yg_m)
    return out2d.reshape(x.shape)
